# Initial kernel scaffold; baseline (speedup 1.0000x reference)
import functools

import jax
import jax.numpy as jnp
from jax import lax
from jax.experimental import pallas as pl
from jax.experimental.pallas import tpu as pltpu

_T, _D, _H, _E, _K = 2048, 1024, 4096, 8, 2
_TK = _T * _K
_BT = 256
_SPAD = _TK + _E * _BT
_NB = _SPAD // _BT
_BH = 1024
_NH = _H // _BH


def _gelu_exact(x):
    return 0.5 * x * (1.0 + lax.erf(x * 0.7071067811865476))


def _router_body(x_ref, wg_ref, bg_ref, p_ref, i_ref):
    y = jnp.dot(x_ref[...], wg_ref[...], preferred_element_type=jnp.float32)
    y = y + bg_ref[...]
    m = jnp.max(y, axis=-1, keepdims=True)
    ex = jnp.exp(y - m)
    s = jnp.sum(ex, axis=-1, keepdims=True)
    p = ex / s
    cols = lax.broadcasted_iota(jnp.int32, y.shape, 1)
    p1 = jnp.max(p, axis=-1, keepdims=True)
    i1 = jnp.min(jnp.where(p == p1, cols, _E), axis=-1, keepdims=True)
    pm = jnp.where(cols == i1, -1.0, p)
    p2 = jnp.max(pm, axis=-1, keepdims=True)
    i2 = jnp.min(jnp.where(pm == p2, cols, _E), axis=-1, keepdims=True)
    p_ref[...] = jnp.concatenate([p1, p2], axis=1)
    i_ref[...] = jnp.concatenate([i1, i2], axis=1)


def _router(x, Wg, bg):
    return pl.pallas_call(
        _router_body,
        out_shape=(jax.ShapeDtypeStruct((_T, _K), jnp.float32),
                   jax.ShapeDtypeStruct((_T, _K), jnp.int32)),
    )(x, Wg, bg.reshape(1, _E))


def _gemm_body(eid_ref, act_ref, xg_ref, w1_ref, b1_ref, w2_ref, b2_ref,
               ps_ref, out_ref):
    b = pl.program_id(0)
    nh = pl.program_id(1)

    @pl.when(nh == 0)
    def _init():
        out_ref[...] = jnp.zeros_like(out_ref)

    @pl.when(act_ref[b] == 1)
    def _compute():
        h = jnp.dot(xg_ref[...], w1_ref[0], preferred_element_type=jnp.float32)
        h = _gelu_exact(h + b1_ref[...])
        out_ref[...] += jnp.dot(h, w2_ref[0], preferred_element_type=jnp.float32)

    @pl.when(nh == _NH - 1)
    def _finish():
        out_ref[...] = (out_ref[...] + b2_ref[...]) * ps_ref[:, 0:1]


def _grouped_gemm(eid, act, xg, W1, b1, W2, b2, ps):
    grid_spec = pltpu.PrefetchScalarGridSpec(
        num_scalar_prefetch=2,
        grid=(_NB, _NH),
        in_specs=[
            pl.BlockSpec((_BT, _D), lambda b, nh, eid, act: (b, 0)),
            pl.BlockSpec((1, _D, _BH), lambda b, nh, eid, act: (eid[b], 0, nh)),
            pl.BlockSpec((1, _BH), lambda b, nh, eid, act: (eid[b], nh)),
            pl.BlockSpec((1, _BH, _D), lambda b, nh, eid, act: (eid[b], nh, 0)),
            pl.BlockSpec((1, _D), lambda b, nh, eid, act: (eid[b], 0)),
            pl.BlockSpec((_BT, 128), lambda b, nh, eid, act: (b, 0)),
        ],
        out_specs=pl.BlockSpec((_BT, _D), lambda b, nh, eid, act: (b, 0)),
    )
    return pl.pallas_call(
        _gemm_body,
        grid_spec=grid_spec,
        out_shape=jax.ShapeDtypeStruct((_SPAD, _D), jnp.float32),
        compiler_params=pltpu.CompilerParams(
            dimension_semantics=("arbitrary", "arbitrary")),
    )(eid, act, xg, W1, b1, W2, b2, ps)


def _bookkeeping(idx, probs):
    idx_f = idx.reshape(-1)
    oh = (idx_f[:, None] == jnp.arange(_E, dtype=jnp.int32)[None, :])
    csum = jnp.cumsum(oh.astype(jnp.int32), axis=0)
    counts = csum[-1]
    rank = jnp.take_along_axis(csum, idx_f[:, None], axis=1)[:, 0] - 1
    padded = ((counts + _BT - 1) // _BT) * _BT
    ends = jnp.cumsum(padded)
    starts = ends - padded
    slot = starts[idx_f] + rank
    block_start = jnp.arange(_NB, dtype=jnp.int32) * _BT
    eid = jnp.searchsorted(ends, block_start, side='right').astype(jnp.int32)
    act = (block_start < ends[-1]).astype(jnp.int32)
    eidc = jnp.minimum(eid, _E - 1)
    prob_slot = jnp.zeros((_SPAD,), jnp.float32).at[slot].set(probs.reshape(-1))
    ps = jnp.broadcast_to(prob_slot[:, None], (_SPAD, 128))
    return slot, eidc, act, ps


def kernel(x, Wg, bg, W1, b1, W2, b2):
    probs, idx = _router(x, Wg, bg)
    slot, eidc, act, ps = _bookkeeping(idx, probs)
    token_of_slot = jnp.zeros((_SPAD,), jnp.int32).at[slot].set(
        jnp.arange(_TK, dtype=jnp.int32) // _K)
    xg = x[token_of_slot]
    og = _grouped_gemm(eidc, act, xg, W1, b1, W2, b2, ps)
    slot2 = slot.reshape(_T, _K)
    out = og[slot2[:, 0]] + og[slot2[:, 1]]
    return out


# speedup vs baseline: 2.4296x; 2.4296x over previous
import functools

import jax
import jax.numpy as jnp
from jax import lax
from jax.experimental import pallas as pl
from jax.experimental.pallas import tpu as pltpu

_T, _D, _H, _E, _K = 2048, 1024, 4096, 8, 2
_TK = _T * _K
_BT = 256
_SPAD = _TK + _E * _BT
_NB = _SPAD // _BT
_BH = 1024
_NH = _H // _BH


def _gelu_exact(x):
    return 0.5 * x * (1.0 + lax.erf(x * 0.7071067811865476))


def _router_body(x_ref, wg_ref, bg_ref, p_ref, i_ref):
    y = jnp.dot(x_ref[...], wg_ref[...], preferred_element_type=jnp.float32)
    y = y + bg_ref[...]
    m = jnp.max(y, axis=-1, keepdims=True)
    ex = jnp.exp(y - m)
    s = jnp.sum(ex, axis=-1, keepdims=True)
    p = ex / s
    cols = lax.broadcasted_iota(jnp.int32, y.shape, 1)
    p1 = jnp.max(p, axis=-1, keepdims=True)
    i1 = jnp.min(jnp.where(p == p1, cols, _E), axis=-1, keepdims=True)
    pm = jnp.where(cols == i1, -1.0, p)
    p2 = jnp.max(pm, axis=-1, keepdims=True)
    i2 = jnp.min(jnp.where(pm == p2, cols, _E), axis=-1, keepdims=True)
    p_ref[...] = jnp.concatenate([p1, p2], axis=1)
    i_ref[...] = jnp.concatenate([i1, i2], axis=1)


def _router(x, Wg, bg):
    return pl.pallas_call(
        _router_body,
        out_shape=(jax.ShapeDtypeStruct((_T, _K), jnp.float32),
                   jax.ShapeDtypeStruct((_T, _K), jnp.int32)),
    )(x, Wg, bg.reshape(1, _E))


def _gemm_body(eid_ref, act_ref, xg_ref, w1_ref, b1_ref, w2_ref, b2_ref,
               ps_ref, out_ref):
    b = pl.program_id(0)
    nh = pl.program_id(1)

    @pl.when(nh == 0)
    def _init():
        out_ref[...] = jnp.zeros_like(out_ref)

    @pl.when(act_ref[b] == 1)
    def _compute():
        h = jnp.dot(xg_ref[...], w1_ref[0], preferred_element_type=jnp.float32)
        h = _gelu_exact(h + b1_ref[0, 0])
        out_ref[...] += jnp.dot(h, w2_ref[0], preferred_element_type=jnp.float32)

    @pl.when(nh == _NH - 1)
    def _finish():
        out_ref[...] = (out_ref[...] + b2_ref[0]) * ps_ref[:, 0:1]


def _grouped_gemm(eid, act, xg, W1, b1, W2, b2, ps):
    grid_spec = pltpu.PrefetchScalarGridSpec(
        num_scalar_prefetch=2,
        grid=(_NB, _NH),
        in_specs=[
            pl.BlockSpec((_BT, _D), lambda b, nh, eid, act: (b, 0)),
            pl.BlockSpec((1, _D, _BH), lambda b, nh, eid, act: (eid[b], 0, nh)),
            pl.BlockSpec((1, 1, 1, _BH), lambda b, nh, eid, act: (eid[b], nh, 0, 0)),
            pl.BlockSpec((1, _BH, _D), lambda b, nh, eid, act: (eid[b], nh, 0)),
            pl.BlockSpec((1, 1, _D), lambda b, nh, eid, act: (eid[b], 0, 0)),
            pl.BlockSpec((_BT, 128), lambda b, nh, eid, act: (b, 0)),
        ],
        out_specs=pl.BlockSpec((_BT, _D), lambda b, nh, eid, act: (b, 0)),
    )
    return pl.pallas_call(
        _gemm_body,
        grid_spec=grid_spec,
        out_shape=jax.ShapeDtypeStruct((_SPAD, _D), jnp.float32),
        compiler_params=pltpu.CompilerParams(
            dimension_semantics=("arbitrary", "arbitrary")),
    )(eid, act, xg, W1, b1.reshape(_E, _NH, 1, _BH), W2, b2.reshape(_E, 1, _D),
      ps)


def _bookkeeping(idx, probs):
    idx_f = idx.reshape(-1)
    oh = (idx_f[:, None] == jnp.arange(_E, dtype=jnp.int32)[None, :])
    csum = jnp.cumsum(oh.astype(jnp.int32), axis=0)
    counts = csum[-1]
    rank = jnp.take_along_axis(csum, idx_f[:, None], axis=1)[:, 0] - 1
    padded = ((counts + _BT - 1) // _BT) * _BT
    ends = jnp.cumsum(padded)
    starts = ends - padded
    slot = starts[idx_f] + rank
    block_start = jnp.arange(_NB, dtype=jnp.int32) * _BT
    eid = jnp.searchsorted(ends, block_start, side='right').astype(jnp.int32)
    act = (block_start < ends[-1]).astype(jnp.int32)
    eidc = jnp.minimum(eid, _E - 1)
    prob_slot = jnp.zeros((_SPAD,), jnp.float32).at[slot].set(probs.reshape(-1))
    ps = jnp.broadcast_to(prob_slot[:, None], (_SPAD, 128))
    return slot, eidc, act, ps


def kernel(x, Wg, bg, W1, b1, W2, b2):
    probs, idx = _router(x, Wg, bg)
    slot, eidc, act, ps = _bookkeeping(idx, probs)
    token_of_slot = jnp.zeros((_SPAD,), jnp.int32).at[slot].set(
        jnp.arange(_TK, dtype=jnp.int32) // _K)
    xg = x[token_of_slot]
    og = _grouped_gemm(eidc, act, xg, W1, b1, W2, b2, ps)
    slot2 = slot.reshape(_T, _K)
    out = og[slot2[:, 0]] + og[slot2[:, 1]]
    return out


# speedup vs baseline: 2.8604x; 1.1773x over previous
import functools

import jax
import jax.numpy as jnp
from jax import lax
from jax.experimental import pallas as pl
from jax.experimental.pallas import tpu as pltpu
from jax.experimental.pallas import tpu_sc as plsc

_T, _D, _H, _E, _K = 2048, 1024, 4096, 8, 2
_TK = _T * _K
_BT = 256
_SPAD = _TK + _E * _BT
_NB = _SPAD // _BT
_BH = 1024
_NH = _H // _BH


def _gelu_exact(x):
    return 0.5 * x * (1.0 + lax.erf(x * 0.7071067811865476))


def _router_body(x_ref, wg_ref, bg_ref, p_ref, i_ref):
    y = jnp.dot(x_ref[...], wg_ref[...], preferred_element_type=jnp.float32)
    y = y + bg_ref[...]
    m = jnp.max(y, axis=-1, keepdims=True)
    ex = jnp.exp(y - m)
    s = jnp.sum(ex, axis=-1, keepdims=True)
    p = ex / s
    cols = lax.broadcasted_iota(jnp.int32, y.shape, 1)
    p1 = jnp.max(p, axis=-1, keepdims=True)
    i1 = jnp.min(jnp.where(p == p1, cols, _E), axis=-1, keepdims=True)
    pm = jnp.where(cols == i1, -1.0, p)
    p2 = jnp.max(pm, axis=-1, keepdims=True)
    i2 = jnp.min(jnp.where(pm == p2, cols, _E), axis=-1, keepdims=True)
    p_ref[...] = jnp.concatenate([p1, p2], axis=1)
    i_ref[...] = jnp.concatenate([i1, i2], axis=1)


def _router(x, Wg, bg):
    return pl.pallas_call(
        _router_body,
        out_shape=(jax.ShapeDtypeStruct((_T, _K), jnp.float32),
                   jax.ShapeDtypeStruct((_T, _K), jnp.int32)),
    )(x, Wg, bg.reshape(1, _E))


def _gemm_body(eid_ref, act_ref, xg_ref, w1_ref, b1_ref, w2_ref, b2_ref,
               ps_ref, out_ref):
    b = pl.program_id(0)
    nh = pl.program_id(1)

    @pl.when(nh == 0)
    def _init():
        out_ref[...] = jnp.zeros_like(out_ref)

    @pl.when(act_ref[b] == 1)
    def _compute():
        h = jnp.dot(xg_ref[...], w1_ref[0], preferred_element_type=jnp.float32)
        h = _gelu_exact(h + b1_ref[0, 0])
        out_ref[...] += jnp.dot(h, w2_ref[0], preferred_element_type=jnp.float32)

    @pl.when(nh == _NH - 1)
    def _finish():
        out_ref[...] = (out_ref[...] + b2_ref[0]) * ps_ref[:, 0:1]


def _grouped_gemm(eid, act, xg, W1, b1, W2, b2, ps):
    grid_spec = pltpu.PrefetchScalarGridSpec(
        num_scalar_prefetch=2,
        grid=(_NB, _NH),
        in_specs=[
            pl.BlockSpec((_BT, _D), lambda b, nh, eid, act: (b, 0)),
            pl.BlockSpec((1, _D, _BH), lambda b, nh, eid, act: (eid[b], 0, nh)),
            pl.BlockSpec((1, 1, 1, _BH), lambda b, nh, eid, act: (eid[b], nh, 0, 0)),
            pl.BlockSpec((1, _BH, _D), lambda b, nh, eid, act: (eid[b], nh, 0)),
            pl.BlockSpec((1, 1, _D), lambda b, nh, eid, act: (eid[b], 0, 0)),
            pl.BlockSpec((_BT, 128), lambda b, nh, eid, act: (b, 0)),
        ],
        out_specs=pl.BlockSpec((_BT, _D), lambda b, nh, eid, act: (b, 0)),
    )
    return pl.pallas_call(
        _gemm_body,
        grid_spec=grid_spec,
        out_shape=jax.ShapeDtypeStruct((_SPAD, _D), jnp.float32),
        compiler_params=pltpu.CompilerParams(
            dimension_semantics=("arbitrary", "arbitrary")),
    )(eid, act, xg, W1, b1.reshape(_E, _NH, 1, _BH), W2, b2.reshape(_E, 1, _D),
      ps)


_NC, _NS = 2, 16
_NW = _NC * _NS
_TPW = _T // _NW
_CH = 32

_sc_mesh = plsc.VectorSubcoreMesh(core_axis_name="c", subcore_axis_name="s")


def _sc_wid():
    return lax.axis_index("s") * _NC + lax.axis_index("c")


def _sc_dispatch_body(x_hbm, slot3_hbm, xg_hbm, xv, idxv, sem0, sem1):
    wid = _sc_wid()
    base = wid * _TPW
    pltpu.sync_copy(x_hbm.at[pl.ds(base, _TPW)], xv)
    pltpu.sync_copy(slot3_hbm.at[wid], idxv)
    c0 = pltpu.async_copy(xv, xg_hbm.at[idxv.at[0]], sem0)
    c1 = pltpu.async_copy(xv, xg_hbm.at[idxv.at[1]], sem1)
    c0.wait()
    c1.wait()


_sc_dispatch = functools.partial(
    pl.kernel,
    out_type=jax.ShapeDtypeStruct((_SPAD, _D), jnp.float32),
    mesh=_sc_mesh,
    scratch_types=[
        pltpu.VMEM((_TPW, _D), jnp.float32),
        pltpu.VMEM((_K, _TPW), jnp.int32),
        pltpu.SemaphoreType.DMA,
        pltpu.SemaphoreType.DMA,
    ],
)(_sc_dispatch_body)


def _sc_combine_body(og_hbm, slot3_hbm, out_hbm, idxv, r0, r1, sem0, sem1):
    wid = _sc_wid()
    base = wid * _TPW
    pltpu.sync_copy(slot3_hbm.at[wid], idxv)
    for c in range(_TPW // _CH):
        g0 = pltpu.async_copy(og_hbm.at[idxv.at[0, pl.ds(c * _CH, _CH)]],
                              r0, sem0)
        g1 = pltpu.async_copy(og_hbm.at[idxv.at[1, pl.ds(c * _CH, _CH)]],
                              r1, sem1)
        g0.wait()
        g1.wait()

        def _row_add(j, carry):
            for q in range(_D // 16):
                sl = pl.ds(q * 16, 16)
                r0[j, sl] = r0[j, sl] + r1[j, sl]
            return carry

        lax.fori_loop(0, _CH, _row_add, 0)
        pltpu.sync_copy(r0, out_hbm.at[pl.ds(base + c * _CH, _CH)])


_sc_combine = functools.partial(
    pl.kernel,
    out_type=jax.ShapeDtypeStruct((_T, _D), jnp.float32),
    mesh=_sc_mesh,
    scratch_types=[
        pltpu.VMEM((_K, _TPW), jnp.int32),
        pltpu.VMEM((_CH, _D), jnp.float32),
        pltpu.VMEM((_CH, _D), jnp.float32),
        pltpu.SemaphoreType.DMA,
        pltpu.SemaphoreType.DMA,
    ],
)(_sc_combine_body)


def _bookkeeping(idx, probs):
    idx_f = idx.reshape(-1)
    oh = (idx_f[:, None] == jnp.arange(_E, dtype=jnp.int32)[None, :])
    csum = jnp.cumsum(oh.astype(jnp.int32), axis=0)
    counts = csum[-1]
    rank = jnp.take_along_axis(csum, idx_f[:, None], axis=1)[:, 0] - 1
    padded = ((counts + _BT - 1) // _BT) * _BT
    ends = jnp.cumsum(padded)
    starts = ends - padded
    slot = starts[idx_f] + rank
    block_start = jnp.arange(_NB, dtype=jnp.int32) * _BT
    eid = jnp.searchsorted(ends, block_start, side='right').astype(jnp.int32)
    act = (block_start < ends[-1]).astype(jnp.int32)
    eidc = jnp.minimum(eid, _E - 1)
    prob_slot = jnp.zeros((_SPAD,), jnp.float32).at[slot].set(probs.reshape(-1))
    ps = jnp.broadcast_to(prob_slot[:, None], (_SPAD, 128))
    return slot, eidc, act, ps


def kernel(x, Wg, bg, W1, b1, W2, b2):
    probs, idx = _router(x, Wg, bg)
    slot, eidc, act, ps = _bookkeeping(idx, probs)
    slot3 = slot.reshape(_T, _K).T.reshape(_K, _NW, _TPW).transpose(1, 0, 2)
    xg = _sc_dispatch(x, slot3)
    og = _grouped_gemm(eidc, act, xg, W1, b1, W2, b2, ps)
    out = _sc_combine(og, slot3)
    return out


# speedup vs baseline: 3.5029x; 1.2246x over previous
import functools

import jax
import jax.numpy as jnp
from jax import lax
from jax.experimental import pallas as pl
from jax.experimental.pallas import tpu as pltpu
from jax.experimental.pallas import tpu_sc as plsc

_T, _D, _H, _E, _K = 2048, 1024, 4096, 8, 2
_TK = _T * _K
_BT = 768
_NB = 14
_SPAD = _NB * _BT
_BH = 1024
_NH = _H // _BH


def _gelu_exact(x):
    return 0.5 * x * (1.0 + lax.erf(x * 0.7071067811865476))


def _router_body(x_ref, wg_ref, bg_ref, p_ref, i_ref):
    y = jnp.dot(x_ref[...], wg_ref[...], preferred_element_type=jnp.float32)
    y = y + bg_ref[...]
    m = jnp.max(y, axis=-1, keepdims=True)
    ex = jnp.exp(y - m)
    s = jnp.sum(ex, axis=-1, keepdims=True)
    p = ex / s
    cols = lax.broadcasted_iota(jnp.int32, y.shape, 1)
    p1 = jnp.max(p, axis=-1, keepdims=True)
    i1 = jnp.min(jnp.where(p == p1, cols, _E), axis=-1, keepdims=True)
    pm = jnp.where(cols == i1, -1.0, p)
    p2 = jnp.max(pm, axis=-1, keepdims=True)
    i2 = jnp.min(jnp.where(pm == p2, cols, _E), axis=-1, keepdims=True)
    p_ref[...] = jnp.concatenate([p1, p2], axis=1)
    i_ref[...] = jnp.concatenate([i1, i2], axis=1)


def _router(x, Wg, bg):
    return pl.pallas_call(
        _router_body,
        out_shape=(jax.ShapeDtypeStruct((_T, _K), jnp.float32),
                   jax.ShapeDtypeStruct((_T, _K), jnp.int32)),
    )(x, Wg, bg.reshape(1, _E))


def _gemm_body(eid_ref, act_ref, blk_ref, xg_ref, w1_ref, b1_ref, w2_ref,
               b2_ref, ps_ref, out_ref):
    b = pl.program_id(0)
    nh = pl.program_id(1)
    active = act_ref[b] == 1

    @pl.when(jnp.logical_and(active, nh == 0))
    def _init():
        out_ref[...] = jnp.zeros_like(out_ref)

    @pl.when(active)
    def _compute():
        xb = xg_ref[...].astype(jnp.bfloat16)
        h = jnp.dot(xb, w1_ref[0].astype(jnp.bfloat16),
                    preferred_element_type=jnp.float32)
        h = _gelu_exact(h + b1_ref[0, 0])
        out_ref[...] += jnp.dot(h.astype(jnp.bfloat16),
                                w2_ref[0].astype(jnp.bfloat16),
                                preferred_element_type=jnp.float32)

    @pl.when(jnp.logical_and(active, nh == _NH - 1))
    def _finish():
        out_ref[...] = (out_ref[...] + b2_ref[0]) * ps_ref[:, 0:1]


def _grouped_gemm(eid, act, blkmap, xg, W1, b1, W2, b2, ps):
    grid_spec = pltpu.PrefetchScalarGridSpec(
        num_scalar_prefetch=3,
        grid=(_NB, _NH),
        in_specs=[
            pl.BlockSpec((_BT, _D), lambda b, nh, eid, act, blk: (blk[b], 0)),
            pl.BlockSpec((1, _D, _BH),
                         lambda b, nh, eid, act, blk: (eid[b], 0, nh)),
            pl.BlockSpec((1, 1, 1, _BH),
                         lambda b, nh, eid, act, blk: (eid[b], nh, 0, 0)),
            pl.BlockSpec((1, _BH, _D),
                         lambda b, nh, eid, act, blk: (eid[b], nh, 0)),
            pl.BlockSpec((1, 1, _D),
                         lambda b, nh, eid, act, blk: (eid[b], 0, 0)),
            pl.BlockSpec((_BT, 8), lambda b, nh, eid, act, blk: (blk[b], 0)),
        ],
        out_specs=pl.BlockSpec((_BT, _D),
                               lambda b, nh, eid, act, blk: (blk[b], 0)),
    )
    return pl.pallas_call(
        _gemm_body,
        grid_spec=grid_spec,
        out_shape=jax.ShapeDtypeStruct((_SPAD, _D), jnp.float32),
        compiler_params=pltpu.CompilerParams(
            dimension_semantics=("arbitrary", "arbitrary")),
    )(eid, act, blkmap, xg, W1, b1.reshape(_E, _NH, 1, _BH), W2,
      b2.reshape(_E, 1, _D), ps)


_NC, _NS = 2, 16
_NW = _NC * _NS
_TPW = _T // _NW
_CH = 32

@functools.lru_cache(maxsize=None)
def _sc_mesh():
    return plsc.VectorSubcoreMesh(core_axis_name="c", subcore_axis_name="s")


def _sc_wid():
    return lax.axis_index("s") * _NC + lax.axis_index("c")


def _sc_dispatch_body(x_hbm, slot3_hbm, xg_hbm, xv, idxv, sem0, sem1):
    wid = _sc_wid()
    base = wid * _TPW
    pltpu.sync_copy(x_hbm.at[pl.ds(base, _TPW)], xv)
    pltpu.sync_copy(slot3_hbm.at[wid], idxv)
    c0 = pltpu.async_copy(xv, xg_hbm.at[idxv.at[0]], sem0)
    c1 = pltpu.async_copy(xv, xg_hbm.at[idxv.at[1]], sem1)
    c0.wait()
    c1.wait()


@functools.lru_cache(maxsize=None)
def _sc_dispatch_kernel():
    return pl.kernel(
        _sc_dispatch_body,
        out_type=jax.ShapeDtypeStruct((_SPAD, _D), jnp.float32),
        mesh=_sc_mesh(),
        scratch_types=[
            pltpu.VMEM((_TPW, _D), jnp.float32),
            pltpu.VMEM((_K, _TPW), jnp.int32),
            pltpu.SemaphoreType.DMA,
            pltpu.SemaphoreType.DMA,
        ],
    )


def _sc_combine_body(og_hbm, slot3_hbm, out_hbm, idxv, r0, r1, sem0, sem1):
    wid = _sc_wid()
    base = wid * _TPW
    pltpu.sync_copy(slot3_hbm.at[wid], idxv)
    for c in range(_TPW // _CH):
        g0 = pltpu.async_copy(og_hbm.at[idxv.at[0, pl.ds(c * _CH, _CH)]],
                              r0, sem0)
        g1 = pltpu.async_copy(og_hbm.at[idxv.at[1, pl.ds(c * _CH, _CH)]],
                              r1, sem1)
        g0.wait()
        g1.wait()

        def _row_add(j, carry):
            for q in range(_D // 16):
                sl = pl.ds(q * 16, 16)
                r0[j, sl] = r0[j, sl] + r1[j, sl]
            return carry

        lax.fori_loop(0, _CH, _row_add, 0)
        pltpu.sync_copy(r0, out_hbm.at[pl.ds(base + c * _CH, _CH)])


@functools.lru_cache(maxsize=None)
def _sc_combine_kernel():
    return pl.kernel(
        _sc_combine_body,
        out_type=jax.ShapeDtypeStruct((_T, _D), jnp.float32),
        mesh=_sc_mesh(),
        scratch_types=[
            pltpu.VMEM((_K, _TPW), jnp.int32),
            pltpu.VMEM((_CH, _D), jnp.float32),
            pltpu.VMEM((_CH, _D), jnp.float32),
            pltpu.SemaphoreType.DMA,
            pltpu.SemaphoreType.DMA,
        ],
    )


def _bookkeeping(idx, probs):
    idx_f = idx.reshape(-1)
    oh = (idx_f[:, None] == jnp.arange(_E, dtype=jnp.int32)[None, :])
    csum = jnp.cumsum(oh.astype(jnp.int32), axis=0)
    counts = csum[-1]
    rank = jnp.take_along_axis(csum, idx_f[:, None], axis=1)[:, 0] - 1
    padded = ((counts + _BT - 1) // _BT) * _BT
    ends = jnp.cumsum(padded)
    starts = ends - padded
    slot = starts[idx_f] + rank
    block_start = jnp.arange(_NB, dtype=jnp.int32) * _BT
    eid = jnp.searchsorted(ends, block_start, side='right').astype(jnp.int32)
    act = (block_start < ends[-1]).astype(jnp.int32)
    nact = jnp.sum(act)
    eidc = jnp.where(act == 1, jnp.minimum(eid, _E - 1),
                     jnp.minimum(eid[nact - 1], _E - 1))
    blkmap = jnp.minimum(jnp.arange(_NB, dtype=jnp.int32), nact - 1)
    prob_slot = jnp.zeros((_SPAD,), jnp.float32).at[slot].set(probs.reshape(-1))
    ps = jnp.broadcast_to(prob_slot[:, None], (_SPAD, 8))
    return slot, eidc, act, blkmap, ps


def kernel(x, Wg, bg, W1, b1, W2, b2):
    probs, idx = _router(x, Wg, bg)
    slot, eidc, act, blkmap, ps = _bookkeeping(idx, probs)
    slot3 = slot.reshape(_T, _K).T.reshape(_K, _NW, _TPW).transpose(1, 0, 2)
    xg = _sc_dispatch_kernel()(x, slot3)
    og = _grouped_gemm(eidc, act, blkmap, xg, W1, b1, W2, b2, ps)
    out = _sc_combine_kernel()(og, slot3)
    return out


# speedup vs baseline: 3.7589x; 1.0731x over previous
import functools

import jax
import jax.numpy as jnp
from jax import lax
from jax.experimental import pallas as pl
from jax.experimental.pallas import tpu as pltpu
from jax.experimental.pallas import tpu_sc as plsc

_T, _D, _H, _E, _K = 2048, 1024, 4096, 8, 2
_TK = _T * _K
_BT = 768
_NB = 14
_SPAD = _NB * _BT
_BH = 1024
_NH = _H // _BH


def _gelu_exact(x):
    return 0.5 * x * (1.0 + lax.erf(x * 0.7071067811865476))


def _router_body(x_ref, wg_ref, bg_ref, p_ref, i_ref):
    y = jnp.dot(x_ref[...], wg_ref[...], preferred_element_type=jnp.float32)
    y = y + bg_ref[...]
    m = jnp.max(y, axis=-1, keepdims=True)
    ex = jnp.exp(y - m)
    s = jnp.sum(ex, axis=-1, keepdims=True)
    p = ex / s
    cols = lax.broadcasted_iota(jnp.int32, y.shape, 1)
    p1 = jnp.max(p, axis=-1, keepdims=True)
    i1 = jnp.min(jnp.where(p == p1, cols, _E), axis=-1, keepdims=True)
    pm = jnp.where(cols == i1, -1.0, p)
    p2 = jnp.max(pm, axis=-1, keepdims=True)
    i2 = jnp.min(jnp.where(pm == p2, cols, _E), axis=-1, keepdims=True)
    p_ref[...] = jnp.concatenate([p1, p2], axis=1)
    i_ref[...] = jnp.concatenate([i1, i2], axis=1)


def _router(x, Wg, bg):
    return pl.pallas_call(
        _router_body,
        out_shape=(jax.ShapeDtypeStruct((_T, _K), jnp.float32),
                   jax.ShapeDtypeStruct((_T, _K), jnp.int32)),
    )(x, Wg, bg.reshape(1, _E))


def _gemm_body(eid_ref, act_ref, blk_ref, xg_ref, w1_ref, b1_ref, w2_ref,
               b2_ref, out_ref):
    b = pl.program_id(0)
    nh = pl.program_id(1)
    active = act_ref[b] == 1

    @pl.when(jnp.logical_and(active, nh == 0))
    def _init():
        out_ref[...] = jnp.broadcast_to(b2_ref[0], out_ref.shape)

    @pl.when(active)
    def _compute():
        xb = xg_ref[...].astype(jnp.bfloat16)
        h = jnp.dot(xb, w1_ref[0].astype(jnp.bfloat16),
                    preferred_element_type=jnp.float32)
        h = _gelu_exact(h + b1_ref[0, 0])
        out_ref[...] += jnp.dot(h.astype(jnp.bfloat16),
                                w2_ref[0].astype(jnp.bfloat16),
                                preferred_element_type=jnp.float32)


def _grouped_gemm(eid, act, blkmap, xg, W1, b1, W2, b2):
    grid_spec = pltpu.PrefetchScalarGridSpec(
        num_scalar_prefetch=3,
        grid=(_NB, _NH),
        in_specs=[
            pl.BlockSpec((_BT, _D), lambda b, nh, eid, act, blk: (blk[b], 0)),
            pl.BlockSpec((1, _D, _BH),
                         lambda b, nh, eid, act, blk: (eid[b], 0, nh)),
            pl.BlockSpec((1, 1, 1, _BH),
                         lambda b, nh, eid, act, blk: (eid[b], nh, 0, 0)),
            pl.BlockSpec((1, _BH, _D),
                         lambda b, nh, eid, act, blk: (eid[b], nh, 0)),
            pl.BlockSpec((1, 1, _D),
                         lambda b, nh, eid, act, blk: (eid[b], 0, 0)),
        ],
        out_specs=pl.BlockSpec((_BT, _D),
                               lambda b, nh, eid, act, blk: (blk[b], 0)),
    )
    return pl.pallas_call(
        _gemm_body,
        grid_spec=grid_spec,
        out_shape=jax.ShapeDtypeStruct((_SPAD, _D), jnp.float32),
        compiler_params=pltpu.CompilerParams(
            dimension_semantics=("arbitrary", "arbitrary")),
    )(eid, act, blkmap, xg, W1, b1.reshape(_E, _NH, 1, _BH), W2,
      b2.reshape(_E, 1, _D))


_NC, _NS = 2, 16
_NW = _NC * _NS
_TPW = _T // _NW
_CH = 32

@functools.lru_cache(maxsize=None)
def _sc_mesh():
    return plsc.VectorSubcoreMesh(core_axis_name="c", subcore_axis_name="s")


def _sc_wid():
    return lax.axis_index("s") * _NC + lax.axis_index("c")


def _sc_dispatch_body(x_hbm, slot3_hbm, xg_hbm, xv, idxv, sem0, sem1):
    wid = _sc_wid()
    base = wid * _TPW
    pltpu.sync_copy(x_hbm.at[pl.ds(base, _TPW)], xv)
    pltpu.sync_copy(slot3_hbm.at[wid], idxv)
    c0 = pltpu.async_copy(xv, xg_hbm.at[idxv.at[0]], sem0)
    c1 = pltpu.async_copy(xv, xg_hbm.at[idxv.at[1]], sem1)
    c0.wait()
    c1.wait()


@functools.lru_cache(maxsize=None)
def _sc_dispatch_kernel():
    return pl.kernel(
        _sc_dispatch_body,
        out_type=jax.ShapeDtypeStruct((_SPAD, _D), jnp.float32),
        mesh=_sc_mesh(),
        scratch_types=[
            pltpu.VMEM((_TPW, _D), jnp.float32),
            pltpu.VMEM((_K, _TPW), jnp.int32),
            pltpu.SemaphoreType.DMA,
            pltpu.SemaphoreType.DMA,
        ],
    )


def _sc_combine_body(og_hbm, slot3_hbm, pb_hbm, out_hbm, idxv, pv, r0, r1,
                     sem0, sem1):
    wid = _sc_wid()
    base = wid * _TPW
    pltpu.sync_copy(slot3_hbm.at[wid], idxv)
    pltpu.sync_copy(pb_hbm.at[pl.ds(base, _TPW)], pv)
    for c in range(_TPW // _CH):
        g0 = pltpu.async_copy(og_hbm.at[idxv.at[0, pl.ds(c * _CH, _CH)]],
                              r0, sem0)
        g1 = pltpu.async_copy(og_hbm.at[idxv.at[1, pl.ds(c * _CH, _CH)]],
                              r1, sem1)
        g0.wait()
        g1.wait()

        def _row_comb(j, carry):
            p0 = pv[c * _CH + j, pl.ds(0, 16)]
            p1 = pv[c * _CH + j, pl.ds(16, 16)]
            for q in range(_D // 16):
                sl = pl.ds(q * 16, 16)
                r0[j, sl] = r0[j, sl] * p0 + r1[j, sl] * p1
            return carry

        lax.fori_loop(0, _CH, _row_comb, 0)
        pltpu.sync_copy(r0, out_hbm.at[pl.ds(base + c * _CH, _CH)])


@functools.lru_cache(maxsize=None)
def _sc_combine_kernel():
    return pl.kernel(
        _sc_combine_body,
        out_type=jax.ShapeDtypeStruct((_T, _D), jnp.float32),
        mesh=_sc_mesh(),
        scratch_types=[
            pltpu.VMEM((_K, _TPW), jnp.int32),
            pltpu.VMEM((_TPW, _K * 16), jnp.float32),
            pltpu.VMEM((_CH, _D), jnp.float32),
            pltpu.VMEM((_CH, _D), jnp.float32),
            pltpu.SemaphoreType.DMA,
            pltpu.SemaphoreType.DMA,
        ],
    )


def _bookkeeping(idx, probs):
    idx_f = idx.reshape(-1)
    oh = (idx_f[:, None] == jnp.arange(_E, dtype=jnp.int32)[None, :])
    csum = jnp.cumsum(oh.astype(jnp.int32), axis=0)
    counts = csum[-1]
    rank = jnp.take_along_axis(csum, idx_f[:, None], axis=1)[:, 0] - 1
    padded = ((counts + _BT - 1) // _BT) * _BT
    ends = jnp.cumsum(padded)
    starts = ends - padded
    slot = starts[idx_f] + rank
    block_start = jnp.arange(_NB, dtype=jnp.int32) * _BT
    eid = jnp.searchsorted(ends, block_start, side='right').astype(jnp.int32)
    act = (block_start < ends[-1]).astype(jnp.int32)
    nact = jnp.sum(act)
    eidc = jnp.where(act == 1, jnp.minimum(eid, _E - 1),
                     jnp.minimum(eid[nact - 1], _E - 1))
    blkmap = jnp.minimum(jnp.arange(_NB, dtype=jnp.int32), nact - 1)
    return slot, eidc, act, blkmap


def kernel(x, Wg, bg, W1, b1, W2, b2):
    probs, idx = _router(x, Wg, bg)
    slot, eidc, act, blkmap = _bookkeeping(idx, probs)
    slot3 = slot.reshape(_T, _K).T.reshape(_K, _NW, _TPW).transpose(1, 0, 2)
    pb = jnp.broadcast_to(probs[:, :, None], (_T, _K, 16)).reshape(_T, _K * 16)
    xg = _sc_dispatch_kernel()(x, slot3)
    og = _grouped_gemm(eidc, act, blkmap, xg, W1, b1, W2, b2)
    out = _sc_combine_kernel()(og, slot3, pb)
    return out


# speedup vs baseline: 4.0303x; 1.0722x over previous
import functools

import jax
import jax.numpy as jnp
from jax import lax
from jax.experimental import pallas as pl
from jax.experimental.pallas import tpu as pltpu
from jax.experimental.pallas import tpu_sc as plsc

_T, _D, _H, _E, _K = 2048, 1024, 4096, 8, 2
_TK = _T * _K
_BT = 768
_NB = 14
_SPAD = _NB * _BT
_BH = 1024
_NH = _H // _BH


def _gelu_exact(x):
    return 0.5 * x * (1.0 + lax.erf(x * 0.7071067811865476))


_CC = 128
_NCH = _T // _CC


def _router_body(x_ref, wg_ref, bg_ref, p_ref, slot_ref, ends_ref):
    y = jnp.dot(x_ref[...], wg_ref[...], preferred_element_type=jnp.float32)
    y = y + bg_ref[...]
    m = jnp.max(y, axis=-1, keepdims=True)
    ex = jnp.exp(y - m)
    s = jnp.sum(ex, axis=-1, keepdims=True)
    p = ex / s
    cols = lax.broadcasted_iota(jnp.int32, y.shape, 1)
    p1 = jnp.max(p, axis=-1, keepdims=True)
    i1 = jnp.min(jnp.where(p == p1, cols, _E), axis=-1, keepdims=True)
    pm = jnp.where(cols == i1, -1.0, p)
    p2 = jnp.max(pm, axis=-1, keepdims=True)
    i2 = jnp.min(jnp.where(pm == p2, cols, _E), axis=-1, keepdims=True)
    p_ref[...] = jnp.concatenate([p1, p2], axis=1)

    oh1 = (cols == i1).astype(jnp.float32)
    oh2 = (cols == i2).astype(jnp.float32)
    ohs = oh1 + oh2
    r_i = lax.broadcasted_iota(jnp.int32, (_CC, _CC), 0)
    c_i = lax.broadcasted_iota(jnp.int32, (_CC, _CC), 1)
    tri = (c_i < r_i).astype(jnp.float32)
    chunks = []
    off = jnp.zeros((1, _E), jnp.float32)
    for c in range(_NCH):
        ch = ohs[c * _CC:(c + 1) * _CC, :]
        loc = jnp.dot(tri, ch, preferred_element_type=jnp.float32)
        chunks.append(loc + off)
        off = off + jnp.sum(ch, axis=0, keepdims=True)
    prefix = jnp.concatenate(chunks, axis=0)
    counts = off
    counts_i = counts.astype(jnp.int32)
    padded = (((counts_i + _BT - 1) // _BT) * _BT).astype(jnp.float32)
    tri_e = (lax.broadcasted_iota(jnp.int32, (_E, _E), 0)
             <= lax.broadcasted_iota(jnp.int32, (_E, _E), 1)).astype(jnp.float32)
    ends = jnp.dot(padded, tri_e, preferred_element_type=jnp.float32)
    starts = ends - padded
    rank0 = jnp.sum(prefix * oh1, axis=1, keepdims=True)
    rank1 = jnp.sum(prefix * oh2, axis=1, keepdims=True)
    start0 = jnp.sum(starts * oh1, axis=1, keepdims=True)
    start1 = jnp.sum(starts * oh2, axis=1, keepdims=True)
    slot01 = jnp.concatenate([start0 + rank0, start1 + rank1], axis=1)
    slot_ref[...] = slot01.astype(jnp.int32)
    ends_ref[...] = ends.astype(jnp.int32)


def _router(x, Wg, bg):
    return pl.pallas_call(
        _router_body,
        out_shape=(jax.ShapeDtypeStruct((_T, _K), jnp.float32),
                   jax.ShapeDtypeStruct((_T, _K), jnp.int32),
                   jax.ShapeDtypeStruct((1, _E), jnp.int32)),
    )(x, Wg, bg.reshape(1, _E))


def _gemm_body(eid_ref, act_ref, blk_ref, xg_ref, w1_ref, b1_ref, w2_ref,
               b2_ref, out_ref):
    b = pl.program_id(0)
    nh = pl.program_id(1)
    active = act_ref[b] == 1

    @pl.when(jnp.logical_and(active, nh == 0))
    def _init():
        out_ref[...] = jnp.broadcast_to(b2_ref[0], out_ref.shape)

    @pl.when(active)
    def _compute():
        xb = xg_ref[...].astype(jnp.bfloat16)
        h = jnp.dot(xb, w1_ref[0].astype(jnp.bfloat16),
                    preferred_element_type=jnp.float32)
        h = _gelu_exact(h + b1_ref[0, 0])
        out_ref[...] += jnp.dot(h.astype(jnp.bfloat16),
                                w2_ref[0].astype(jnp.bfloat16),
                                preferred_element_type=jnp.float32)


def _grouped_gemm(eid, act, blkmap, xg, W1, b1, W2, b2):
    grid_spec = pltpu.PrefetchScalarGridSpec(
        num_scalar_prefetch=3,
        grid=(_NB, _NH),
        in_specs=[
            pl.BlockSpec((_BT, _D), lambda b, nh, eid, act, blk: (blk[b], 0)),
            pl.BlockSpec((1, _D, _BH),
                         lambda b, nh, eid, act, blk: (eid[b], 0, nh)),
            pl.BlockSpec((1, 1, 1, _BH),
                         lambda b, nh, eid, act, blk: (eid[b], nh, 0, 0)),
            pl.BlockSpec((1, _BH, _D),
                         lambda b, nh, eid, act, blk: (eid[b], nh, 0)),
            pl.BlockSpec((1, 1, _D),
                         lambda b, nh, eid, act, blk: (eid[b], 0, 0)),
        ],
        out_specs=pl.BlockSpec((_BT, _D),
                               lambda b, nh, eid, act, blk: (blk[b], 0)),
    )
    return pl.pallas_call(
        _gemm_body,
        grid_spec=grid_spec,
        out_shape=jax.ShapeDtypeStruct((_SPAD, _D), jnp.float32),
        compiler_params=pltpu.CompilerParams(
            dimension_semantics=("arbitrary", "arbitrary")),
    )(eid, act, blkmap, xg, W1, b1.reshape(_E, _NH, 1, _BH), W2,
      b2.reshape(_E, 1, _D))


_NC, _NS = 2, 16
_NW = _NC * _NS
_TPW = _T // _NW
_CH = 32

@functools.lru_cache(maxsize=None)
def _sc_mesh():
    return plsc.VectorSubcoreMesh(core_axis_name="c", subcore_axis_name="s")


def _sc_wid():
    return lax.axis_index("s") * _NC + lax.axis_index("c")


def _sc_dispatch_body(x_hbm, slot3_hbm, xg_hbm, xv, idxv, sem0, sem1):
    wid = _sc_wid()
    base = wid * _TPW
    pltpu.sync_copy(x_hbm.at[pl.ds(base, _TPW)], xv)
    pltpu.sync_copy(slot3_hbm.at[wid], idxv)
    c0 = pltpu.async_copy(xv, xg_hbm.at[idxv.at[0]], sem0)
    c1 = pltpu.async_copy(xv, xg_hbm.at[idxv.at[1]], sem1)
    c0.wait()
    c1.wait()


@functools.lru_cache(maxsize=None)
def _sc_dispatch_kernel():
    return pl.kernel(
        _sc_dispatch_body,
        out_type=jax.ShapeDtypeStruct((_SPAD, _D), jnp.float32),
        mesh=_sc_mesh(),
        scratch_types=[
            pltpu.VMEM((_TPW, _D), jnp.float32),
            pltpu.VMEM((_K, _TPW), jnp.int32),
            pltpu.SemaphoreType.DMA,
            pltpu.SemaphoreType.DMA,
        ],
    )


def _sc_combine_body(og_hbm, slot3_hbm, pb_hbm, out_hbm, idxv, pv, r0, r1,
                     sem0, sem1):
    wid = _sc_wid()
    base = wid * _TPW
    pltpu.sync_copy(slot3_hbm.at[wid], idxv)
    pltpu.sync_copy(pb_hbm.at[pl.ds(base, _TPW)], pv)
    for c in range(_TPW // _CH):
        g0 = pltpu.async_copy(og_hbm.at[idxv.at[0, pl.ds(c * _CH, _CH)]],
                              r0, sem0)
        g1 = pltpu.async_copy(og_hbm.at[idxv.at[1, pl.ds(c * _CH, _CH)]],
                              r1, sem1)
        g0.wait()
        g1.wait()

        def _row_comb(j, carry):
            p0 = pv[c * _CH + j, pl.ds(0, 16)]
            p1 = pv[c * _CH + j, pl.ds(16, 16)]
            for q in range(_D // 16):
                sl = pl.ds(q * 16, 16)
                r0[j, sl] = r0[j, sl] * p0 + r1[j, sl] * p1
            return carry

        lax.fori_loop(0, _CH, _row_comb, 0)
        pltpu.sync_copy(r0, out_hbm.at[pl.ds(base + c * _CH, _CH)])


@functools.lru_cache(maxsize=None)
def _sc_combine_kernel():
    return pl.kernel(
        _sc_combine_body,
        out_type=jax.ShapeDtypeStruct((_T, _D), jnp.float32),
        mesh=_sc_mesh(),
        scratch_types=[
            pltpu.VMEM((_K, _TPW), jnp.int32),
            pltpu.VMEM((_TPW, _K * 16), jnp.float32),
            pltpu.VMEM((_CH, _D), jnp.float32),
            pltpu.VMEM((_CH, _D), jnp.float32),
            pltpu.SemaphoreType.DMA,
            pltpu.SemaphoreType.DMA,
        ],
    )


def _block_meta(ends):
    ends = ends.reshape(_E)
    block_start = jnp.arange(_NB, dtype=jnp.int32) * _BT
    eid = jnp.sum((ends[None, :] <= block_start[:, None]).astype(jnp.int32),
                  axis=1)
    act = (block_start < ends[-1]).astype(jnp.int32)
    nact = jnp.sum(act)
    eidc = jnp.where(act == 1, jnp.minimum(eid, _E - 1),
                     jnp.minimum(eid[nact - 1], _E - 1))
    blkmap = jnp.minimum(jnp.arange(_NB, dtype=jnp.int32), nact - 1)
    return eidc, act, blkmap


def kernel(x, Wg, bg, W1, b1, W2, b2):
    probs, slot2, ends = _router(x, Wg, bg)
    eidc, act, blkmap = _block_meta(ends)
    slot3 = slot2.T.reshape(_K, _NW, _TPW).transpose(1, 0, 2)
    pb = jnp.broadcast_to(probs[:, :, None], (_T, _K, 16)).reshape(_T, _K * 16)
    xg = _sc_dispatch_kernel()(x, slot3)
    og = _grouped_gemm(eidc, act, blkmap, xg, W1, b1, W2, b2)
    out = _sc_combine_kernel()(og, slot3, pb)
    return out


# speedup vs baseline: 4.1053x; 1.0186x over previous
import functools

import jax
import jax.numpy as jnp
from jax import lax
from jax.experimental import pallas as pl
from jax.experimental.pallas import tpu as pltpu
from jax.experimental.pallas import tpu_sc as plsc

_T, _D, _H, _E, _K = 2048, 1024, 4096, 8, 2
_TK = _T * _K
_BT = 768
_NB = 14
_SPAD = _NB * _BT
_BH = 1024
_NH = _H // _BH


def _gelu_exact(x):
    return 0.5 * x * (1.0 + lax.erf(x * 0.7071067811865476))


_CC = 128
_NCH = _T // _CC


def _router_body(x_ref, wg_ref, bg_ref, p_ref, slot_ref, ends_ref):
    y = jnp.dot(x_ref[...], wg_ref[...], preferred_element_type=jnp.float32)
    y = y + bg_ref[...]
    m = jnp.max(y, axis=-1, keepdims=True)
    ex = jnp.exp(y - m)
    s = jnp.sum(ex, axis=-1, keepdims=True)
    p = ex / s
    cols = lax.broadcasted_iota(jnp.int32, y.shape, 1)
    p1 = jnp.max(p, axis=-1, keepdims=True)
    i1 = jnp.min(jnp.where(p == p1, cols, _E), axis=-1, keepdims=True)
    pm = jnp.where(cols == i1, -1.0, p)
    p2 = jnp.max(pm, axis=-1, keepdims=True)
    i2 = jnp.min(jnp.where(pm == p2, cols, _E), axis=-1, keepdims=True)
    p_ref[...] = jnp.concatenate([jnp.broadcast_to(p1, (_T, 16)),
                                  jnp.broadcast_to(p2, (_T, 16))], axis=1)

    oh1 = (cols == i1).astype(jnp.float32)
    oh2 = (cols == i2).astype(jnp.float32)
    ohs = oh1 + oh2
    r_i = lax.broadcasted_iota(jnp.int32, (_CC, _CC), 0)
    c_i = lax.broadcasted_iota(jnp.int32, (_CC, _CC), 1)
    tri = (c_i < r_i).astype(jnp.float32)
    chunks = []
    off = jnp.zeros((1, _E), jnp.float32)
    for c in range(_NCH):
        ch = ohs[c * _CC:(c + 1) * _CC, :]
        loc = jnp.dot(tri, ch, preferred_element_type=jnp.float32)
        chunks.append(loc + off)
        off = off + jnp.sum(ch, axis=0, keepdims=True)
    prefix = jnp.concatenate(chunks, axis=0)
    counts = off
    counts_i = counts.astype(jnp.int32)
    padded = (((counts_i + _BT - 1) // _BT) * _BT).astype(jnp.float32)
    tri_e = (lax.broadcasted_iota(jnp.int32, (_E, _E), 0)
             <= lax.broadcasted_iota(jnp.int32, (_E, _E), 1)).astype(jnp.float32)
    ends = jnp.dot(padded, tri_e, preferred_element_type=jnp.float32)
    starts = ends - padded
    rank0 = jnp.sum(prefix * oh1, axis=1, keepdims=True)
    rank1 = jnp.sum(prefix * oh2, axis=1, keepdims=True)
    start0 = jnp.sum(starts * oh1, axis=1, keepdims=True)
    start1 = jnp.sum(starts * oh2, axis=1, keepdims=True)
    slot01 = jnp.concatenate([start0 + rank0, start1 + rank1], axis=1)
    slot_ref[...] = slot01.astype(jnp.int32)
    ends_ref[...] = ends.astype(jnp.int32)


def _router(x, Wg, bg):
    return pl.pallas_call(
        _router_body,
        out_shape=(jax.ShapeDtypeStruct((_T, _K * 16), jnp.float32),
                   jax.ShapeDtypeStruct((_T, _K), jnp.int32),
                   jax.ShapeDtypeStruct((1, _E), jnp.int32)),
    )(x, Wg, bg.reshape(1, _E))


def _gemm_body(eid_ref, act_ref, blk_ref, xg_ref, w1_ref, b1_ref, w2_ref,
               b2_ref, out_ref):
    b = pl.program_id(0)
    nh = pl.program_id(1)
    active = act_ref[b] == 1

    @pl.when(jnp.logical_and(active, nh == 0))
    def _init():
        out_ref[...] = jnp.broadcast_to(b2_ref[0], out_ref.shape)

    @pl.when(active)
    def _compute():
        xb = xg_ref[...].astype(jnp.bfloat16)
        h = jnp.dot(xb, w1_ref[0].astype(jnp.bfloat16),
                    preferred_element_type=jnp.float32)
        h = _gelu_exact(h + b1_ref[0, 0])
        out_ref[...] += jnp.dot(h.astype(jnp.bfloat16),
                                w2_ref[0].astype(jnp.bfloat16),
                                preferred_element_type=jnp.float32)


def _grouped_gemm(eid, act, blkmap, xg, W1, b1, W2, b2):
    grid_spec = pltpu.PrefetchScalarGridSpec(
        num_scalar_prefetch=3,
        grid=(_NB, _NH),
        in_specs=[
            pl.BlockSpec((_BT, _D), lambda b, nh, eid, act, blk: (blk[b], 0)),
            pl.BlockSpec((1, _D, _BH),
                         lambda b, nh, eid, act, blk: (eid[b], 0, nh)),
            pl.BlockSpec((1, 1, 1, _BH),
                         lambda b, nh, eid, act, blk: (eid[b], nh, 0, 0)),
            pl.BlockSpec((1, _BH, _D),
                         lambda b, nh, eid, act, blk: (eid[b], nh, 0)),
            pl.BlockSpec((1, 1, _D),
                         lambda b, nh, eid, act, blk: (eid[b], 0, 0)),
        ],
        out_specs=pl.BlockSpec((_BT, _D),
                               lambda b, nh, eid, act, blk: (blk[b], 0)),
    )
    return pl.pallas_call(
        _gemm_body,
        grid_spec=grid_spec,
        out_shape=jax.ShapeDtypeStruct((_SPAD, _D), jnp.float32),
        compiler_params=pltpu.CompilerParams(
            dimension_semantics=("arbitrary", "arbitrary")),
    )(eid, act, blkmap, xg, W1, b1.reshape(_E, _NH, 1, _BH), W2,
      b2.reshape(_E, 1, _D))


_NC, _NS = 2, 16
_NW = _NC * _NS
_TPW = _T // _NW
_CH = 16

@functools.lru_cache(maxsize=None)
def _sc_mesh():
    return plsc.VectorSubcoreMesh(core_axis_name="c", subcore_axis_name="s")


def _sc_wid():
    return lax.axis_index("s") * _NC + lax.axis_index("c")


def _sc_dispatch_body(x_hbm, slot3_hbm, xg_hbm, xv, idxv, sem0, sem1):
    wid = _sc_wid()
    base = wid * _TPW
    pltpu.sync_copy(x_hbm.at[pl.ds(base, _TPW)], xv)
    pltpu.sync_copy(slot3_hbm.at[wid], idxv)
    c0 = pltpu.async_copy(xv, xg_hbm.at[idxv.at[0]], sem0)
    c1 = pltpu.async_copy(xv, xg_hbm.at[idxv.at[1]], sem1)
    c0.wait()
    c1.wait()


@functools.lru_cache(maxsize=None)
def _sc_dispatch_kernel():
    return pl.kernel(
        _sc_dispatch_body,
        out_type=jax.ShapeDtypeStruct((_SPAD, _D), jnp.float32),
        mesh=_sc_mesh(),
        scratch_types=[
            pltpu.VMEM((_TPW, _D), jnp.float32),
            pltpu.VMEM((_K, _TPW), jnp.int32),
            pltpu.SemaphoreType.DMA,
            pltpu.SemaphoreType.DMA,
        ],
    )


def _sc_combine_body(og_hbm, slot3_hbm, pb_hbm, out_hbm, idxv, pv,
                     r0a, r1a, r0b, r1b, gs0a, gs1a, gs0b, gs1b, wsa, wsb):
    wid = _sc_wid()
    base = wid * _TPW
    pltpu.sync_copy(slot3_hbm.at[wid], idxv)
    pltpu.sync_copy(pb_hbm.at[pl.ds(base, _TPW)], pv)
    nchk = _TPW // _CH
    bufs = [(r0a, r1a, gs0a, gs1a, wsa), (r0b, r1b, gs0b, gs1b, wsb)]

    def _gather(c):
        r0, r1, s0, s1, _ = bufs[c % 2]
        g0 = pltpu.async_copy(og_hbm.at[idxv.at[0, pl.ds(c * _CH, _CH)]],
                              r0, s0)
        g1 = pltpu.async_copy(og_hbm.at[idxv.at[1, pl.ds(c * _CH, _CH)]],
                              r1, s1)
        return g0, g1

    pend_g = {0: _gather(0)}
    pend_w = {}
    for c in range(nchk):
        r0, r1, _, _, ws = bufs[c % 2]
        if c - 1 in pend_w:
            pend_w.pop(c - 1).wait()
        if c + 1 < nchk:
            pend_g[c + 1] = _gather(c + 1)
        g0, g1 = pend_g.pop(c)
        g0.wait()
        g1.wait()

        def _row_comb(j, carry):
            p0 = pv[c * _CH + j, pl.ds(0, 16)]
            p1 = pv[c * _CH + j, pl.ds(16, 16)]
            for q in range(_D // 16):
                sl = pl.ds(q * 16, 16)
                r0[j, sl] = r0[j, sl] * p0 + r1[j, sl] * p1
            return carry

        lax.fori_loop(0, _CH, _row_comb, 0)
        pend_w[c] = pltpu.async_copy(r0, out_hbm.at[pl.ds(base + c * _CH,
                                                          _CH)], ws)
    for c in sorted(pend_w):
        pend_w.pop(c).wait()


@functools.lru_cache(maxsize=None)
def _sc_combine_kernel():
    return pl.kernel(
        _sc_combine_body,
        out_type=jax.ShapeDtypeStruct((_T, _D), jnp.float32),
        mesh=_sc_mesh(),
        scratch_types=[
            pltpu.VMEM((_K, _TPW), jnp.int32),
            pltpu.VMEM((_TPW, _K * 16), jnp.float32),
            pltpu.VMEM((_CH, _D), jnp.float32),
            pltpu.VMEM((_CH, _D), jnp.float32),
            pltpu.VMEM((_CH, _D), jnp.float32),
            pltpu.VMEM((_CH, _D), jnp.float32),
            pltpu.SemaphoreType.DMA,
            pltpu.SemaphoreType.DMA,
            pltpu.SemaphoreType.DMA,
            pltpu.SemaphoreType.DMA,
            pltpu.SemaphoreType.DMA,
            pltpu.SemaphoreType.DMA,
        ],
    )


def _block_meta(ends):
    ends = ends.reshape(_E)
    block_start = jnp.arange(_NB, dtype=jnp.int32) * _BT
    eid = jnp.sum((ends[None, :] <= block_start[:, None]).astype(jnp.int32),
                  axis=1)
    act = (block_start < ends[-1]).astype(jnp.int32)
    nact = jnp.sum(act)
    eidc = jnp.where(act == 1, jnp.minimum(eid, _E - 1),
                     jnp.minimum(eid[nact - 1], _E - 1))
    blkmap = jnp.minimum(jnp.arange(_NB, dtype=jnp.int32), nact - 1)
    return eidc, act, blkmap


def kernel(x, Wg, bg, W1, b1, W2, b2):
    pb, slot2, ends = _router(x, Wg, bg)
    eidc, act, blkmap = _block_meta(ends)
    slot3 = slot2.T.reshape(_K, _NW, _TPW).transpose(1, 0, 2)
    xg = _sc_dispatch_kernel()(x, slot3)
    og = _grouped_gemm(eidc, act, blkmap, xg, W1, b1, W2, b2)
    out = _sc_combine_kernel()(og, slot3, pb)
    return out


# speedup vs baseline: 4.2759x; 1.0415x over previous
import functools

import jax
import jax.numpy as jnp
from jax import lax
from jax.experimental import pallas as pl
from jax.experimental.pallas import tpu as pltpu
from jax.experimental.pallas import tpu_sc as plsc

_T, _D, _H, _E, _K = 2048, 1024, 4096, 8, 2
_TK = _T * _K
_BT = 768
_NB = 13
_SPAD = _NB * _BT
_BH = 1024
_NH = _H // _BH


def _gelu_exact(x):
    return 0.5 * x * (1.0 + lax.erf(x * 0.7071067811865476))


_CC = 128
_NCH = _T // _CC


def _router_body(x_ref, wg_ref, bg_ref, p_ref, slot_ref, ends_ref):
    y = jnp.dot(x_ref[...], wg_ref[...], preferred_element_type=jnp.float32)
    y = y + bg_ref[...]
    m = jnp.max(y, axis=-1, keepdims=True)
    ex = jnp.exp(y - m)
    s = jnp.sum(ex, axis=-1, keepdims=True)
    p = ex / s
    cols = lax.broadcasted_iota(jnp.int32, y.shape, 1)
    p1 = jnp.max(p, axis=-1, keepdims=True)
    i1 = jnp.min(jnp.where(p == p1, cols, _E), axis=-1, keepdims=True)
    pm = jnp.where(cols == i1, -1.0, p)
    p2 = jnp.max(pm, axis=-1, keepdims=True)
    i2 = jnp.min(jnp.where(pm == p2, cols, _E), axis=-1, keepdims=True)
    p_ref[...] = jnp.concatenate([jnp.broadcast_to(p1, (_T, 16)),
                                  jnp.broadcast_to(p2, (_T, 16))], axis=1)

    oh1 = (cols == i1).astype(jnp.float32)
    oh2 = (cols == i2).astype(jnp.float32)
    ohs = oh1 + oh2
    r_i = lax.broadcasted_iota(jnp.int32, (_CC, _CC), 0)
    c_i = lax.broadcasted_iota(jnp.int32, (_CC, _CC), 1)
    tri = (c_i < r_i).astype(jnp.float32)
    chunks = []
    off = jnp.zeros((1, _E), jnp.float32)
    for c in range(_NCH):
        ch = ohs[c * _CC:(c + 1) * _CC, :]
        loc = jnp.dot(tri, ch, preferred_element_type=jnp.float32)
        chunks.append(loc + off)
        off = off + jnp.sum(ch, axis=0, keepdims=True)
    prefix = jnp.concatenate(chunks, axis=0)
    counts = off
    counts_i = counts.astype(jnp.int32)
    padded = (((counts_i + _BT - 1) // _BT) * _BT).astype(jnp.float32)
    tri_e = (lax.broadcasted_iota(jnp.int32, (_E, _E), 0)
             <= lax.broadcasted_iota(jnp.int32, (_E, _E), 1)).astype(jnp.float32)
    ends = jnp.dot(padded, tri_e, preferred_element_type=jnp.float32)
    starts = ends - padded
    rank0 = jnp.sum(prefix * oh1, axis=1, keepdims=True)
    rank1 = jnp.sum(prefix * oh2, axis=1, keepdims=True)
    start0 = jnp.sum(starts * oh1, axis=1, keepdims=True)
    start1 = jnp.sum(starts * oh2, axis=1, keepdims=True)
    slot01 = jnp.concatenate([start0 + rank0, start1 + rank1], axis=1)
    slot_ref[...] = slot01.astype(jnp.int32)
    ends_ref[...] = ends.astype(jnp.int32)


def _router(x, Wg, bg):
    return pl.pallas_call(
        _router_body,
        out_shape=(jax.ShapeDtypeStruct((_T, _K * 16), jnp.float32),
                   jax.ShapeDtypeStruct((_T, _K), jnp.int32),
                   jax.ShapeDtypeStruct((1, _E), jnp.int32)),
    )(x, Wg, bg.reshape(1, _E))


def _gemm_body(eid_ref, act_ref, blk_ref, xg_ref, w1_ref, b1_ref, w2_ref,
               b2_ref, out_ref):
    b = pl.program_id(0)
    nh = pl.program_id(1)
    active = act_ref[b] == 1

    @pl.when(jnp.logical_and(active, nh == 0))
    def _init():
        out_ref[...] = jnp.broadcast_to(b2_ref[0], out_ref.shape)

    @pl.when(active)
    def _compute():
        xb = xg_ref[...].astype(jnp.bfloat16)
        h = jnp.dot(xb, w1_ref[0].astype(jnp.bfloat16),
                    preferred_element_type=jnp.float32)
        h = _gelu_exact(h + b1_ref[0, 0])
        out_ref[...] += jnp.dot(h.astype(jnp.bfloat16),
                                w2_ref[0].astype(jnp.bfloat16),
                                preferred_element_type=jnp.float32)


def _grouped_gemm(eid, act, blkmap, xg, W1, b1, W2, b2):
    grid_spec = pltpu.PrefetchScalarGridSpec(
        num_scalar_prefetch=3,
        grid=(_NB, _NH),
        in_specs=[
            pl.BlockSpec((_BT, _D), lambda b, nh, eid, act, blk: (blk[b], 0)),
            pl.BlockSpec((1, _D, _BH),
                         lambda b, nh, eid, act, blk: (eid[b], 0, nh)),
            pl.BlockSpec((1, 1, 1, _BH),
                         lambda b, nh, eid, act, blk: (eid[b], nh, 0, 0)),
            pl.BlockSpec((1, _BH, _D),
                         lambda b, nh, eid, act, blk: (eid[b], nh, 0)),
            pl.BlockSpec((1, 1, _D),
                         lambda b, nh, eid, act, blk: (eid[b], 0, 0)),
        ],
        out_specs=pl.BlockSpec((_BT, _D),
                               lambda b, nh, eid, act, blk: (blk[b], 0)),
    )
    return pl.pallas_call(
        _gemm_body,
        grid_spec=grid_spec,
        out_shape=jax.ShapeDtypeStruct((_SPAD, _D), jnp.float32),
        compiler_params=pltpu.CompilerParams(
            dimension_semantics=("arbitrary", "arbitrary")),
    )(eid, act, blkmap, xg, W1, b1.reshape(_E, _NH, 1, _BH), W2,
      b2.reshape(_E, 1, _D))


_NC, _NS = 2, 16
_NW = _NC * _NS
_TPW = _T // _NW
_CH = 16

@functools.lru_cache(maxsize=None)
def _sc_mesh():
    return plsc.VectorSubcoreMesh(core_axis_name="c", subcore_axis_name="s")


def _sc_wid():
    return lax.axis_index("s") * _NC + lax.axis_index("c")


_HB = _TPW // 2


def _sc_dispatch_body(x_hbm, slot3_hbm, xg_hbm, xva, xvb, idxv,
                      sla, slb, s0a, s1a, s0b, s1b):
    wid = _sc_wid()
    base = wid * _TPW
    la = pltpu.async_copy(x_hbm.at[pl.ds(base, _HB)], xva, sla)
    lb = pltpu.async_copy(x_hbm.at[pl.ds(base + _HB, _HB)], xvb, slb)
    pltpu.sync_copy(slot3_hbm.at[wid], idxv)
    la.wait()
    c0a = pltpu.async_copy(xva, xg_hbm.at[idxv.at[0, pl.ds(0, _HB)]], s0a)
    c1a = pltpu.async_copy(xva, xg_hbm.at[idxv.at[1, pl.ds(0, _HB)]], s1a)
    lb.wait()
    c0b = pltpu.async_copy(xvb, xg_hbm.at[idxv.at[0, pl.ds(_HB, _HB)]], s0b)
    c1b = pltpu.async_copy(xvb, xg_hbm.at[idxv.at[1, pl.ds(_HB, _HB)]], s1b)
    c0a.wait()
    c1a.wait()
    c0b.wait()
    c1b.wait()


@functools.lru_cache(maxsize=None)
def _sc_dispatch_kernel():
    return pl.kernel(
        _sc_dispatch_body,
        out_type=jax.ShapeDtypeStruct((_SPAD, _D), jnp.float32),
        mesh=_sc_mesh(),
        scratch_types=[
            pltpu.VMEM((_HB, _D), jnp.float32),
            pltpu.VMEM((_HB, _D), jnp.float32),
            pltpu.VMEM((_K, _TPW), jnp.int32),
            pltpu.SemaphoreType.DMA,
            pltpu.SemaphoreType.DMA,
            pltpu.SemaphoreType.DMA,
            pltpu.SemaphoreType.DMA,
            pltpu.SemaphoreType.DMA,
            pltpu.SemaphoreType.DMA,
        ],
    )


def _sc_combine_body(og_hbm, slot3_hbm, pb_hbm, out_hbm, idxv, pv,
                     r0a, r1a, r0b, r1b, gs0a, gs1a, gs0b, gs1b, wsa, wsb):
    wid = _sc_wid()
    base = wid * _TPW
    pltpu.sync_copy(slot3_hbm.at[wid], idxv)
    pltpu.sync_copy(pb_hbm.at[pl.ds(base, _TPW)], pv)
    nchk = _TPW // _CH
    bufs = [(r0a, r1a, gs0a, gs1a, wsa), (r0b, r1b, gs0b, gs1b, wsb)]

    def _gather(c):
        r0, r1, s0, s1, _ = bufs[c % 2]
        g0 = pltpu.async_copy(og_hbm.at[idxv.at[0, pl.ds(c * _CH, _CH)]],
                              r0, s0)
        g1 = pltpu.async_copy(og_hbm.at[idxv.at[1, pl.ds(c * _CH, _CH)]],
                              r1, s1)
        return g0, g1

    pend_g = {0: _gather(0)}
    pend_w = {}
    for c in range(nchk):
        r0, r1, _, _, ws = bufs[c % 2]
        if c - 1 in pend_w:
            pend_w.pop(c - 1).wait()
        if c + 1 < nchk:
            pend_g[c + 1] = _gather(c + 1)
        g0, g1 = pend_g.pop(c)
        g0.wait()
        g1.wait()

        def _row_comb(j, carry):
            p0 = pv[c * _CH + j, pl.ds(0, 16)]
            p1 = pv[c * _CH + j, pl.ds(16, 16)]
            for q in range(_D // 16):
                sl = pl.ds(q * 16, 16)
                r0[j, sl] = r0[j, sl] * p0 + r1[j, sl] * p1
            return carry

        lax.fori_loop(0, _CH, _row_comb, 0)
        pend_w[c] = pltpu.async_copy(r0, out_hbm.at[pl.ds(base + c * _CH,
                                                          _CH)], ws)
    for c in sorted(pend_w):
        pend_w.pop(c).wait()


@functools.lru_cache(maxsize=None)
def _sc_combine_kernel():
    return pl.kernel(
        _sc_combine_body,
        out_type=jax.ShapeDtypeStruct((_T, _D), jnp.float32),
        mesh=_sc_mesh(),
        scratch_types=[
            pltpu.VMEM((_K, _TPW), jnp.int32),
            pltpu.VMEM((_TPW, _K * 16), jnp.float32),
            pltpu.VMEM((_CH, _D), jnp.float32),
            pltpu.VMEM((_CH, _D), jnp.float32),
            pltpu.VMEM((_CH, _D), jnp.float32),
            pltpu.VMEM((_CH, _D), jnp.float32),
            pltpu.SemaphoreType.DMA,
            pltpu.SemaphoreType.DMA,
            pltpu.SemaphoreType.DMA,
            pltpu.SemaphoreType.DMA,
            pltpu.SemaphoreType.DMA,
            pltpu.SemaphoreType.DMA,
        ],
    )


def _block_meta(ends):
    ends = ends.reshape(_E)
    block_start = jnp.arange(_NB, dtype=jnp.int32) * _BT
    eid = jnp.sum((ends[None, :] <= block_start[:, None]).astype(jnp.int32),
                  axis=1)
    act = (block_start < ends[-1]).astype(jnp.int32)
    nact = jnp.sum(act)
    eidc = jnp.where(act == 1, jnp.minimum(eid, _E - 1),
                     jnp.minimum(eid[nact - 1], _E - 1))
    blkmap = jnp.minimum(jnp.arange(_NB, dtype=jnp.int32), nact - 1)
    return eidc, act, blkmap


def kernel(x, Wg, bg, W1, b1, W2, b2):
    pb, slot2, ends = _router(x, Wg, bg)
    eidc, act, blkmap = _block_meta(ends)
    slot3 = slot2.T.reshape(_K, _NW, _TPW).transpose(1, 0, 2)
    xg = _sc_dispatch_kernel()(x, slot3)
    og = _grouped_gemm(eidc, act, blkmap, xg, W1, b1, W2, b2)
    out = _sc_combine_kernel()(og, slot3, pb)
    return out
